# Initial kernel scaffold; baseline (speedup 1.0000x reference)
import functools

import jax
import jax.numpy as jnp
from jax import lax
from jax.experimental import pallas as pl
from jax.experimental.pallas import tpu as pltpu
from jax.experimental.pallas import tpu_sc as plsc

_V = 100000
_S = 200
_B = 1024
_LANES = 16
_CHUNK = 112
_NC_PER_PHRASE = 2
_SPAD = _CHUNK * _NC_PER_PHRASE
_KPC = _CHUNK // _LANES
_VPAD = _V + 8
_NW = 32
_PPW = _B // _NW


def _body(ids_hbm, wpad_hbm, bias_hbm, out_hbm,
          ids_v, vals_v, scratch_v, outbuf_v, bias_v, sem):
    wid = lax.axis_index("s") * 2 + lax.axis_index("c")
    base = wid * _PPW
    pltpu.sync_copy(ids_hbm.at[pl.ds(base, _PPW)], ids_v)
    pltpu.sync_copy(bias_hbm, bias_v)
    lane = lax.iota(jnp.int32, _LANES)
    bvec = bias_v[...]

    def phrase(p, carry):
        copies = [
            pltpu.async_copy(wpad_hbm.at[ids_v.at[p, c]], vals_v.at[c], sem)
            for c in range(_NC_PER_PHRASE)
        ]
        tagbase = p * _SPAD
        for c in range(_NC_PER_PHRASE):
            for k in range(_KPC):
                ids16 = ids_v[p, c, pl.ds(k * _LANES, _LANES)]
                tags16 = lane + (tagbase + c * _CHUNK + k * _LANES)
                plsc.store_scatter(scratch_v, [ids16], tags16)
        for cp in copies:
            cp.wait()
        acc = jnp.zeros((_LANES,), jnp.float32)
        for c in range(_NC_PER_PHRASE):
            for k in range(_KPC):
                ids16 = ids_v[p, c, pl.ds(k * _LANES, _LANES)]
                tags16 = lane + (tagbase + c * _CHUNK + k * _LANES)
                r16 = plsc.load_gather(scratch_v, [ids16])
                v16 = vals_v[c, pl.ds(k * _LANES, _LANES)]
                acc = acc + jnp.where(r16 == tags16, v16, 0.0)
        tot = jnp.sum(acc)
        out16 = jnp.full((_LANES,), tot, jnp.float32) + bvec
        plsc.store_scatter(outbuf_v, [jnp.full((_LANES,), p, jnp.int32)],
                           out16, mask=lane == 0)
        return carry

    lax.fori_loop(0, _PPW, phrase, 0)
    pltpu.sync_copy(outbuf_v, out_hbm.at[pl.ds(base, _PPW)])


_mnb_sc = functools.partial(
    pl.kernel,
    out_type=jax.ShapeDtypeStruct((_B,), jnp.float32),
    mesh=plsc.VectorSubcoreMesh(core_axis_name="c", subcore_axis_name="s"),
    scratch_types=[
        pltpu.VMEM((_PPW, _NC_PER_PHRASE, _CHUNK), jnp.int32),
        pltpu.VMEM((_NC_PER_PHRASE, _CHUNK), jnp.float32),
        pltpu.VMEM((_VPAD,), jnp.int32),
        pltpu.VMEM((_PPW,), jnp.float32),
        pltpu.VMEM((_LANES,), jnp.float32),
        pltpu.SemaphoreType.DMA,
    ],
)(_body)


@jax.jit
def kernel(text, W, b):
    ids = text.astype(jnp.int32).T
    pad = jnp.full((_B, _SPAD - _S), _V, jnp.int32)
    ids3 = jnp.concatenate([ids, pad], axis=1).reshape(_B, _NC_PER_PHRASE, _CHUNK)
    wpad = jnp.concatenate(
        [W[0].astype(jnp.float32), jnp.zeros((_VPAD - _V,), jnp.float32)])
    bias16 = jnp.broadcast_to(b.astype(jnp.float32), (_LANES,))
    out = _mnb_sc(ids3, wpad, bias16)
    return out.reshape(_B, 1)


# speedup vs baseline: 10.4877x; 10.4877x over previous
import functools

import jax
import jax.numpy as jnp
from jax import lax
from jax.experimental import pallas as pl
from jax.experimental.pallas import tpu as pltpu
from jax.experimental.pallas import tpu_sc as plsc

_V = 100000
_S = 200
_B = 1024
_LANES = 16
_CHUNK = 112
_NC_PER_PHRASE = 2
_SPAD = _CHUNK * _NC_PER_PHRASE
_KPC = _CHUNK // _LANES
_VPAD = _V + 8
_NW = 32
_PPW = _B // _NW


def _body(ids_hbm, wpad_hbm, bias_hbm, out_hbm,
          ids_v, vals_v, scratch_v, outbuf_v, bias_v, sem):
    wid = lax.axis_index("s") * 2 + lax.axis_index("c")
    base = wid * _PPW
    pltpu.sync_copy(ids_hbm.at[pl.ds(base, _PPW)], ids_v)
    pltpu.sync_copy(bias_hbm, bias_v)
    lane = lax.iota(jnp.int32, _LANES)
    bvec = bias_v[...]

    def phrase(p, carry):
        copies = [
            pltpu.async_copy(wpad_hbm.at[ids_v.at[p, c]], vals_v.at[c], sem)
            for c in range(_NC_PER_PHRASE)
        ]
        tagbase = p * _SPAD
        for c in range(_NC_PER_PHRASE):
            for k in range(_KPC):
                ids16 = ids_v[p, c, pl.ds(k * _LANES, _LANES)]
                tags16 = lane + (tagbase + c * _CHUNK + k * _LANES)
                plsc.store_scatter(scratch_v, [ids16], tags16)
        for cp in copies:
            cp.wait()
        acc = jnp.zeros((_LANES,), jnp.float32)
        for c in range(_NC_PER_PHRASE):
            for k in range(_KPC):
                ids16 = ids_v[p, c, pl.ds(k * _LANES, _LANES)]
                tags16 = lane + (tagbase + c * _CHUNK + k * _LANES)
                r16 = plsc.load_gather(scratch_v, [ids16])
                v16 = vals_v[c, pl.ds(k * _LANES, _LANES)]
                acc = acc + jnp.where(r16 == tags16, v16, 0.0)
        tot = jnp.sum(acc)
        out16 = jnp.full((_LANES,), tot, jnp.float32) + bvec
        plsc.store_scatter(outbuf_v, [jnp.full((_LANES,), p, jnp.int32)],
                           out16, mask=lane == 0)
        return carry

    lax.fori_loop(0, _PPW, phrase, 0)
    pltpu.sync_copy(outbuf_v, out_hbm.at[pl.ds(base, _PPW)])


_mnb_sc = functools.partial(
    pl.kernel,
    out_type=jax.ShapeDtypeStruct((_B,), jnp.float32),
    mesh=plsc.VectorSubcoreMesh(core_axis_name="c", subcore_axis_name="s"),
    compiler_params=pltpu.CompilerParams(needs_layout_passes=False),
    scratch_types=[
        pltpu.VMEM((_PPW, _NC_PER_PHRASE, _CHUNK), jnp.int32),
        pltpu.VMEM((_NC_PER_PHRASE, _CHUNK), jnp.float32),
        pltpu.VMEM((_VPAD,), jnp.int32),
        pltpu.VMEM((_PPW,), jnp.float32),
        pltpu.VMEM((_LANES,), jnp.float32),
        pltpu.SemaphoreType.DMA,
    ],
)(_body)


@jax.jit
def kernel(text, W, b):
    ids = text.astype(jnp.int32).T
    pad = jnp.full((_B, _SPAD - _S), _V, jnp.int32)
    ids3 = jnp.concatenate([ids, pad], axis=1).reshape(_B, _NC_PER_PHRASE, _CHUNK)
    wpad = jnp.concatenate(
        [W[0].astype(jnp.float32), jnp.zeros((_VPAD - _V,), jnp.float32)])
    bias16 = jnp.broadcast_to(b.astype(jnp.float32), (_LANES,))
    out = _mnb_sc(ids3, wpad, bias16)
    return out.reshape(_B, 1)
